# baseline (speedup 1.0000x reference)
import functools

import jax
import jax.numpy as jnp
from jax import lax
from jax.experimental import pallas as pl
from jax.experimental.pallas import tpu as pltpu
from jax.experimental.pallas import tpu_sc as plsc

B = 16384
H = 200
V = 1002
VPAD = 1008
E = 4
L = 16
NC = 2
NS = 16
NW = NC * NS
ROWS_PER_W = B // NW
CHUNK = 16
NCHUNK = ROWS_PER_W // CHUNK

_mesh = plsc.VectorSubcoreMesh(core_axis_name="c", subcore_axis_name="s")


@functools.partial(
    pl.kernel,
    mesh=_mesh,
    out_type=jax.ShapeDtypeStruct((B,), jnp.float32),
    compiler_params=pltpu.CompilerParams(needs_layout_passes=False),
    scratch_types=[
        pltpu.VMEM((E * VPAD,), jnp.float32),
        pltpu.VMEM((E * L,), jnp.float32),
        pltpu.VMEM((VPAD,), jnp.float32),
        pltpu.VMEM((CHUNK, H), jnp.int32),
        pltpu.VMEM((CHUNK, H), jnp.int32),
        pltpu.VMEM((CHUNK, H), jnp.float32),
        pltpu.VMEM((CHUNK, H), jnp.float32),
        pltpu.VMEM((ROWS_PER_W,), jnp.float32),
        pltpu.SemaphoreType.DMA,
        pltpu.SemaphoreType.DMA,
    ],
)
def _dwc_kernel(idx_hbm, cnt_hbm, tab_hbm, w_hbm, out_hbm,
                tab_v, w_v, comb_v, idx0_v, idx1_v, cnt0_v, cnt1_v,
                res_v, sem0, sem1):
    cid = lax.axis_index("c")
    sid = lax.axis_index("s")
    wid = sid * NC + cid
    lanes = lax.iota(jnp.int32, L)

    pltpu.sync_copy(tab_hbm, tab_v)
    pltpu.sync_copy(w_hbm, w_v)

    wsplat = [w_v[pl.ds(e * L, L)] for e in range(E)]

    def fold_body(k, carry):
        base = k * L
        acc = jnp.zeros((L,), jnp.float32)
        for e in range(E):
            acc = acc + tab_v[pl.ds(e * VPAD + base, L)] * wsplat[e]
        comb_v[pl.ds(base, L)] = acc
        return carry

    lax.fori_loop(0, VPAD // L, fold_body, 0)

    row0 = wid * ROWS_PER_W
    UNROLL = 8

    def start_fetch(c, ibuf, cbuf, sem):
        r = row0 + jnp.minimum(c, NCHUNK - 1) * CHUNK
        pltpu.async_copy(idx_hbm.at[pl.ds(r, CHUNK), :], ibuf, sem)
        pltpu.async_copy(cnt_hbm.at[pl.ds(r, CHUNK), :], cbuf, sem)

    def wait_fetch(ibuf, cbuf, sem):
        pltpu.make_async_copy(idx_hbm.at[pl.ds(0, CHUNK), :], ibuf, sem).wait()
        pltpu.make_async_copy(cnt_hbm.at[pl.ds(0, CHUNK), :], cbuf, sem).wait()

    def compute(c, ibuf, cbuf):
        def d_body(dd, acc):
            d0 = dd * UNROLL
            for j in range(UNROLL):
                dvec = jnp.full((L,), d0 + j, jnp.int32)
                ii = plsc.load_gather(ibuf, [lanes, dvec])
                cc = plsc.load_gather(cbuf, [lanes, dvec])
                vv = plsc.load_gather(comb_v, [ii])
                acc = acc + cc * vv
            return acc

        acc = lax.fori_loop(0, H // UNROLL, d_body,
                            jnp.zeros((L,), jnp.float32))
        res_v[pl.ds(c * CHUNK, CHUNK)] = acc

    start_fetch(0, idx0_v, cnt0_v, sem0)

    def pair_body(c2, carry):
        c_even = c2 * 2
        start_fetch(c_even + 1, idx1_v, cnt1_v, sem1)
        wait_fetch(idx0_v, cnt0_v, sem0)
        compute(c_even, idx0_v, cnt0_v)
        start_fetch(c_even + 2, idx0_v, cnt0_v, sem0)
        wait_fetch(idx1_v, cnt1_v, sem1)
        compute(c_even + 1, idx1_v, cnt1_v)
        return carry

    lax.fori_loop(0, NCHUNK // 2, pair_body, 0)
    wait_fetch(idx0_v, cnt0_v, sem0)

    pltpu.sync_copy(res_v, out_hbm.at[pl.ds(row0, ROWS_PER_W)])


def kernel(domain_indices, counts, embd_weight, weights):
    tab_t = jnp.zeros((E, VPAD), jnp.float32).at[:, :V].set(embd_weight.T)
    tab_flat = tab_t.reshape(E * VPAD)
    w_bcast = jnp.broadcast_to(weights.reshape(E, 1), (E, L)).reshape(E * L)
    out = _dwc_kernel(domain_indices, counts, tab_flat, w_bcast)
    return out.reshape(B, 1)


# speedup vs baseline: 1.8160x; 1.8160x over previous
import functools

import jax
import jax.numpy as jnp
from jax import lax
from jax.experimental import pallas as pl
from jax.experimental.pallas import tpu as pltpu
from jax.experimental.pallas import tpu_sc as plsc

B = 16384
H = 200
V = 1002
VPAD = 1008
E = 4
L = 16
NC = 2
NS = 16
NW = NC * NS
ROWS_PER_W = B // NW
CHUNK = 16
NCHUNK = ROWS_PER_W // CHUNK
NVEC = 2 * H // L

_mesh = plsc.VectorSubcoreMesh(core_axis_name="c", subcore_axis_name="s")


@functools.partial(
    pl.kernel,
    mesh=_mesh,
    out_type=jax.ShapeDtypeStruct((B,), jnp.float32),
    compiler_params=pltpu.CompilerParams(needs_layout_passes=False),
    scratch_types=[
        pltpu.VMEM((E * VPAD,), jnp.float32),
        pltpu.VMEM((E * L,), jnp.float32),
        pltpu.VMEM((VPAD,), jnp.float32),
        pltpu.VMEM((CHUNK, H), jnp.int32),
        pltpu.VMEM((CHUNK, H), jnp.int32),
        pltpu.VMEM((CHUNK, H), jnp.float32),
        pltpu.VMEM((CHUNK, H), jnp.float32),
        pltpu.VMEM((ROWS_PER_W,), jnp.float32),
        pltpu.SemaphoreType.DMA,
        pltpu.SemaphoreType.DMA,
    ],
)
def _dwc_kernel(idx_hbm, cnt_hbm, tab_hbm, w_hbm, out_hbm,
                tab_v, w_v, comb_v, idx0_v, idx1_v, cnt0_v, cnt1_v,
                res_v, sem0, sem1):
    cid = lax.axis_index("c")
    sid = lax.axis_index("s")
    wid = sid * NC + cid
    lanes = lax.iota(jnp.int32, L)
    lo_half = lanes < 8
    zeros = jnp.zeros((L,), jnp.float32)

    pltpu.sync_copy(tab_hbm, tab_v)
    pltpu.sync_copy(w_hbm, w_v)

    wsplat = [w_v[pl.ds(e * L, L)] for e in range(E)]

    def fold_body(k, carry):
        base = k * L
        acc = zeros
        for e in range(E):
            acc = acc + tab_v[pl.ds(e * VPAD + base, L)] * wsplat[e]
        comb_v[pl.ds(base, L)] = acc
        return carry

    lax.fori_loop(0, VPAD // L, fold_body, 0)

    row0 = wid * ROWS_PER_W

    def start_fetch(c, ibuf, cbuf, sem):
        r = row0 + jnp.minimum(c, NCHUNK - 1) * CHUNK
        pltpu.async_copy(idx_hbm.at[pl.ds(r, CHUNK), :], ibuf, sem)
        pltpu.async_copy(cnt_hbm.at[pl.ds(r, CHUNK), :], cbuf, sem)

    def wait_fetch(ibuf, cbuf, sem):
        pltpu.make_async_copy(idx_hbm.at[pl.ds(0, CHUNK), :], ibuf, sem).wait()
        pltpu.make_async_copy(cnt_hbm.at[pl.ds(0, CHUNK), :], cbuf, sem).wait()

    def compute(c, ibuf, cbuf):
        def pair_body(g, sums):
            ra = g * 2
            rows_a = jnp.full((L,), ra, jnp.int32)
            rows_b = rows_a + 1
            rows_mix = jnp.where(lo_half, rows_a, rows_b)
            cols_mix = jnp.where(lo_half, (H - 8) + lanes, lanes - 8)
            acc_a = zeros
            acc_b = zeros
            for t in range(NVEC):
                if t < 12:
                    rows, cols = rows_a, t * L + lanes
                elif t == 12:
                    rows, cols = rows_mix, cols_mix
                else:
                    rows, cols = rows_b, (t * L - H) + lanes
                ii = plsc.load_gather(ibuf, [rows, cols])
                cc = plsc.load_gather(cbuf, [rows, cols])
                vv = plsc.load_gather(comb_v, [ii])
                prod = cc * vv
                if t < 12:
                    acc_a = acc_a + prod
                elif t == 12:
                    acc_a = acc_a + jnp.where(lo_half, prod, 0.0)
                    acc_b = acc_b + jnp.where(lo_half, 0.0, prod)
                else:
                    acc_b = acc_b + prod
            sa = jnp.sum(acc_a)
            sb = jnp.sum(acc_b)
            sums = jnp.where(lanes == ra, sa, sums)
            sums = jnp.where(lanes == ra + 1, sb, sums)
            return sums

        sums = lax.fori_loop(0, CHUNK // 2, pair_body, zeros)
        res_v[pl.ds(c * CHUNK, CHUNK)] = sums

    start_fetch(0, idx0_v, cnt0_v, sem0)

    def chunk_pair_body(c2, carry):
        c_even = c2 * 2
        start_fetch(c_even + 1, idx1_v, cnt1_v, sem1)
        wait_fetch(idx0_v, cnt0_v, sem0)
        compute(c_even, idx0_v, cnt0_v)
        start_fetch(c_even + 2, idx0_v, cnt0_v, sem0)
        wait_fetch(idx1_v, cnt1_v, sem1)
        compute(c_even + 1, idx1_v, cnt1_v)
        return carry

    lax.fori_loop(0, NCHUNK // 2, chunk_pair_body, 0)
    wait_fetch(idx0_v, cnt0_v, sem0)

    pltpu.sync_copy(res_v, out_hbm.at[pl.ds(row0, ROWS_PER_W)])


def kernel(domain_indices, counts, embd_weight, weights):
    tab_t = jnp.zeros((E, VPAD), jnp.float32).at[:, :V].set(embd_weight.T)
    tab_flat = tab_t.reshape(E * VPAD)
    w_bcast = jnp.broadcast_to(weights.reshape(E, 1), (E, L)).reshape(E * L)
    out = _dwc_kernel(domain_indices, counts, tab_flat, w_bcast)
    return out.reshape(B, 1)


# speedup vs baseline: 3.3794x; 1.8609x over previous
import functools

import jax
import jax.numpy as jnp
from jax import lax
from jax.experimental import pallas as pl
from jax.experimental.pallas import tpu as pltpu
from jax.experimental.pallas import tpu_sc as plsc

B = 16384
H = 200
V = 1002
VPAD = 1008
E = 4
L = 16
NC = 2
NS = 16
NW = NC * NS
ROWS_PER_W = B // NW
CHUNK = 128
NCHUNK = ROWS_PER_W // CHUNK
NLG = CHUNK // L

_mesh = plsc.VectorSubcoreMesh(core_axis_name="c", subcore_axis_name="s")


@functools.partial(
    pl.kernel,
    mesh=_mesh,
    out_type=jax.ShapeDtypeStruct((B,), jnp.float32),
    compiler_params=pltpu.CompilerParams(needs_layout_passes=False),
    scratch_types=[
        pltpu.VMEM((E * VPAD,), jnp.float32),
        pltpu.VMEM((E * L,), jnp.float32),
        pltpu.VMEM((VPAD,), jnp.float32),
        pltpu.VMEM((H, CHUNK), jnp.int32),
        pltpu.VMEM((H, CHUNK), jnp.int32),
        pltpu.VMEM((H, CHUNK), jnp.float32),
        pltpu.VMEM((H, CHUNK), jnp.float32),
        pltpu.VMEM((ROWS_PER_W,), jnp.float32),
        pltpu.SemaphoreType.DMA,
        pltpu.SemaphoreType.DMA,
    ],
)
def _dwc_kernel(idx_hbm, cnt_hbm, tab_hbm, w_hbm, out_hbm,
                tab_v, w_v, comb_v, idx0_v, idx1_v, cnt0_v, cnt1_v,
                res_v, sem0, sem1):
    cid = lax.axis_index("c")
    sid = lax.axis_index("s")
    wid = sid * NC + cid
    lanes = lax.iota(jnp.int32, L)
    zeros = jnp.zeros((L,), jnp.float32)

    pltpu.sync_copy(tab_hbm, tab_v)
    pltpu.sync_copy(w_hbm, w_v)

    wsplat = [w_v[pl.ds(e * L, L)] for e in range(E)]

    def fold_body(k, carry):
        base = k * L
        acc = zeros
        for e in range(E):
            acc = acc + tab_v[pl.ds(e * VPAD + base, L)] * wsplat[e]
        comb_v[pl.ds(base, L)] = acc
        return carry

    lax.fori_loop(0, VPAD // L, fold_body, 0)

    col0 = wid * ROWS_PER_W

    def start_fetch(c, ibuf, cbuf, sem):
        n0 = col0 + jnp.minimum(c, NCHUNK - 1) * CHUNK
        pltpu.async_copy(idx_hbm.at[:, pl.ds(n0, CHUNK)], ibuf, sem)
        pltpu.async_copy(cnt_hbm.at[:, pl.ds(n0, CHUNK)], cbuf, sem)

    def wait_fetch(ibuf, cbuf, sem):
        pltpu.make_async_copy(idx_hbm.at[:, pl.ds(0, CHUNK)], ibuf, sem).wait()
        pltpu.make_async_copy(cnt_hbm.at[:, pl.ds(0, CHUNK)], cbuf, sem).wait()

    lane_cols = [lg * L + lanes for lg in range(NLG)]

    def compute(c, ibuf, cbuf):
        def d_body(d, accs):
            dvec = jnp.full((L,), d, jnp.int32)
            new = []
            for lg in range(NLG):
                ii = plsc.load_gather(ibuf, [dvec, lane_cols[lg]])
                cc = plsc.load_gather(cbuf, [dvec, lane_cols[lg]])
                vv = plsc.load_gather(comb_v, [ii])
                new.append(accs[lg] + cc * vv)
            return tuple(new)

        accs = lax.fori_loop(0, H, d_body, (zeros,) * NLG)
        for lg in range(NLG):
            res_v[pl.ds(c * CHUNK + lg * L, L)] = accs[lg]

    start_fetch(0, idx0_v, cnt0_v, sem0)

    def chunk_pair_body(c2, carry):
        c_even = c2 * 2
        start_fetch(c_even + 1, idx1_v, cnt1_v, sem1)
        wait_fetch(idx0_v, cnt0_v, sem0)
        compute(c_even, idx0_v, cnt0_v)
        start_fetch(c_even + 2, idx0_v, cnt0_v, sem0)
        wait_fetch(idx1_v, cnt1_v, sem1)
        compute(c_even + 1, idx1_v, cnt1_v)
        return carry

    lax.fori_loop(0, NCHUNK // 2, chunk_pair_body, 0)
    wait_fetch(idx0_v, cnt0_v, sem0)

    pltpu.sync_copy(res_v, out_hbm.at[pl.ds(col0, ROWS_PER_W)])


def kernel(domain_indices, counts, embd_weight, weights):
    idx_t = domain_indices.T
    cnt_t = counts.T
    tab_t = jnp.zeros((E, VPAD), jnp.float32).at[:, :V].set(embd_weight.T)
    tab_flat = tab_t.reshape(E * VPAD)
    w_bcast = jnp.broadcast_to(weights.reshape(E, 1), (E, L)).reshape(E * L)
    out = _dwc_kernel(idx_t, cnt_t, tab_flat, w_bcast)
    return out.reshape(B, 1)


# speedup vs baseline: 3.5649x; 1.0549x over previous
import functools

import jax
import jax.numpy as jnp
from jax import lax
from jax.experimental import pallas as pl
from jax.experimental.pallas import tpu as pltpu
from jax.experimental.pallas import tpu_sc as plsc

B = 16384
H = 200
V = 1002
VPAD = 1008
E = 4
L = 16
NC = 2
NS = 16
NW = NC * NS
ROWS_PER_W = B // NW
CHUNK = 128
NCHUNK = ROWS_PER_W // CHUNK
NLG = CHUNK // L

_mesh = plsc.VectorSubcoreMesh(core_axis_name="c", subcore_axis_name="s")


@functools.partial(
    pl.kernel,
    mesh=_mesh,
    out_type=jax.ShapeDtypeStruct((B,), jnp.float32),
    compiler_params=pltpu.CompilerParams(needs_layout_passes=False),
    scratch_types=[
        pltpu.VMEM((E, V), jnp.float32),
        pltpu.VMEM((E * L,), jnp.float32),
        pltpu.VMEM((VPAD,), jnp.float32),
        pltpu.VMEM((H, CHUNK), jnp.int32),
        pltpu.VMEM((H, CHUNK), jnp.int32),
        pltpu.VMEM((H, CHUNK), jnp.float32),
        pltpu.VMEM((H, CHUNK), jnp.float32),
        pltpu.VMEM((ROWS_PER_W,), jnp.float32),
        pltpu.SemaphoreType.DMA,
        pltpu.SemaphoreType.DMA,
    ],
)
def _dwc_kernel(idx_hbm, cnt_hbm, tab_hbm, w_hbm, out_hbm,
                tab_v, w_v, comb_v, idx0_v, idx1_v, cnt0_v, cnt1_v,
                res_v, sem0, sem1):
    cid = lax.axis_index("c")
    sid = lax.axis_index("s")
    wid = sid * NC + cid
    lanes = lax.iota(jnp.int32, L)
    zeros = jnp.zeros((L,), jnp.float32)

    pltpu.sync_copy(tab_hbm, tab_v)
    pltpu.sync_copy(w_hbm, w_v)

    wsplat = [w_v[pl.ds(e * L, L)] for e in range(E)]
    erow = [jnp.full((L,), e, jnp.int32) for e in range(E)]

    def fold_body(k, carry):
        base = k * L
        cols = base + lanes
        acc = zeros
        for e in range(E):
            acc = acc + plsc.load_gather(tab_v, [erow[e], cols]) * wsplat[e]
        comb_v[pl.ds(base, L)] = acc
        return carry

    lax.fori_loop(0, VPAD // L, fold_body, 0)

    col0 = wid * ROWS_PER_W

    def start_fetch(c, ibuf, cbuf, sem):
        n0 = col0 + jnp.minimum(c, NCHUNK - 1) * CHUNK
        pltpu.async_copy(idx_hbm.at[:, pl.ds(n0, CHUNK)], ibuf, sem)
        pltpu.async_copy(cnt_hbm.at[:, pl.ds(n0, CHUNK)], cbuf, sem)

    def wait_fetch(ibuf, cbuf, sem):
        pltpu.make_async_copy(idx_hbm.at[:, pl.ds(0, CHUNK)], ibuf, sem).wait()
        pltpu.make_async_copy(cnt_hbm.at[:, pl.ds(0, CHUNK)], cbuf, sem).wait()

    lane_cols = [lg * L + lanes for lg in range(NLG)]

    DUNROLL = 2

    def compute(c, ibuf, cbuf):
        def d_body(dd, accs):
            new = list(accs)
            for j in range(DUNROLL):
                dvec = jnp.full((L,), dd * DUNROLL + j, jnp.int32)
                for lg in range(NLG):
                    ii = plsc.load_gather(ibuf, [dvec, lane_cols[lg]])
                    cc = plsc.load_gather(cbuf, [dvec, lane_cols[lg]])
                    vv = plsc.load_gather(comb_v, [ii])
                    new[lg] = new[lg] + cc * vv
            return tuple(new)

        accs = lax.fori_loop(0, H // DUNROLL, d_body, (zeros,) * NLG)
        for lg in range(NLG):
            res_v[pl.ds(c * CHUNK + lg * L, L)] = accs[lg]

    start_fetch(0, idx0_v, cnt0_v, sem0)

    def chunk_pair_body(c2, carry):
        c_even = c2 * 2
        start_fetch(c_even + 1, idx1_v, cnt1_v, sem1)
        wait_fetch(idx0_v, cnt0_v, sem0)
        compute(c_even, idx0_v, cnt0_v)
        start_fetch(c_even + 2, idx0_v, cnt0_v, sem0)
        wait_fetch(idx1_v, cnt1_v, sem1)
        compute(c_even + 1, idx1_v, cnt1_v)
        return carry

    lax.fori_loop(0, NCHUNK // 2, chunk_pair_body, 0)
    wait_fetch(idx0_v, cnt0_v, sem0)

    pltpu.sync_copy(res_v, out_hbm.at[pl.ds(col0, ROWS_PER_W)])


def kernel(domain_indices, counts, embd_weight, weights):
    idx_t = domain_indices.T
    cnt_t = counts.T
    tab_t = embd_weight.T
    w_bcast = jnp.broadcast_to(weights.reshape(E, 1), (E, L)).reshape(E * L)
    out = _dwc_kernel(idx_t, cnt_t, tab_t, w_bcast)
    return out.reshape(B, 1)
